# baseline (speedup 1.0000x reference)
import functools

import jax
import jax.numpy as jnp
from jax import lax
from jax.experimental import pallas as pl
from jax.experimental.pallas import tpu as pltpu
from jax.experimental.pallas import tpu_sc as plsc

N = 10000
E = 320000
D = 128
T = 1000
NUM_PRED = 5
HID = 64
M = N + T * NUM_PRED
MPAD = 15360
NW = 32
EPB = 128
NB_E = E // EPB
TPAD = 5120
NBNEW = TPAD // EPB
RPT = MPAD // 16
NPT = MPAD // NW

C0_BPT = 82
C1_BASE = 1312

EV_BASE = 78
EV_SLAB = 79


def _core_split(s):
    lt2 = s < 2
    start = jnp.where(lt2, C1_BASE + s * 76, C1_BASE + 152 + (s - 2) * 74)
    off = jnp.where(lt2, 0, 2)
    npairs = jnp.where(lt2, 38, 37)
    return start - off, off, npairs


def _even_split(wid):
    start = wid * EV_BASE + jnp.minimum(wid, 4)
    cnt = EV_BASE + jnp.where(wid < 4, 1, 0)
    start_copy = jnp.minimum(start, NB_E - EV_SLAB)
    return start_copy, start - start_copy, cnt


@functools.cache
def _make_sc_count():
    mesh = plsc.VectorSubcoreMesh(core_axis_name="c", subcore_axis_name="s")
    return functools.partial(
        pl.kernel,
        mesh=mesh,
        compiler_params=pltpu.CompilerParams(
            needs_layout_passes=False, use_tc_tiling_on_sc=False),
        out_type=jax.ShapeDtypeStruct((NW, MPAD), jnp.float32),
        scratch_types=[
            pltpu.VMEM((EV_SLAB, 2, EPB), jnp.int32),
            pltpu.VMEM((MPAD,), jnp.float32),
        ],
    )(_sc_count_body)


def _sc_count_body(e3_hbm, out_hbm, slab, acc_v):
    c = lax.axis_index("c")
    s = lax.axis_index("s")
    wid = s * 2 + c
    start_copy, off, cnt = _even_split(wid)
    pltpu.sync_copy(e3_hbm.at[pl.ds(start_copy, EV_SLAB)], slab)

    def zero(i, carry):
        acc_v[pl.ds(i * 16, 16)] = jnp.zeros((16,), jnp.float32)
        return carry

    lax.fori_loop(0, MPAD // 16, zero, 0)
    ones = jnp.full((16,), 1.0, jnp.float32)

    def body(i, carry):
        j = off + i
        for k in range(EPB // 16):
            plsc.addupdate_scatter(acc_v, [slab[j, 1, pl.ds(k * 16, 16)]], ones)
        return carry

    lax.fori_loop(0, cnt, body, 0)
    pltpu.sync_copy(acc_v, out_hbm.at[wid])


@functools.cache
def _make_sc_scalar():
    mesh = plsc.VectorSubcoreMesh(core_axis_name="c", subcore_axis_name="s")
    return functools.partial(
        pl.kernel,
        mesh=mesh,
        compiler_params=pltpu.CompilerParams(
            needs_layout_passes=False, use_tc_tiling_on_sc=False),
        out_type=jax.ShapeDtypeStruct((NW, MPAD), jnp.float32),
        scratch_types=[
            pltpu.VMEM((EV_SLAB, 2, EPB), jnp.int32),
            pltpu.VMEM((NBNEW, EPB), jnp.int32),
            pltpu.VMEM((MPAD,), jnp.float32),
            pltpu.VMEM((MPAD,), jnp.float32),
        ],
    )(_sc_scalar_body)


def _sc_scalar_body(g2_hbm, e3_hbm, t3_hbm, out_hbm, slab, t3_v, g2_v, acc_v):
    c = lax.axis_index("c")
    s = lax.axis_index("s")
    wid = s * 2 + c
    start_copy, off, cnt = _even_split(wid)
    pltpu.sync_copy(e3_hbm.at[pl.ds(start_copy, EV_SLAB)], slab)
    pltpu.sync_copy(t3_hbm, t3_v)
    pltpu.sync_copy(g2_hbm, g2_v)

    def zero(i, carry):
        acc_v[pl.ds(i * 16, 16)] = jnp.zeros((16,), jnp.float32)
        return carry

    lax.fori_loop(0, MPAD // 16, zero, 0)

    def body(i, carry):
        j = off + i
        for k in range(EPB // 16):
            sl = pl.ds(k * 16, 16)
            vals = plsc.load_gather(g2_v, [slab[j, 0, sl]])
            plsc.addupdate_scatter(acc_v, [slab[j, 1, sl]], vals)
        return carry

    lax.fori_loop(0, cnt, body, 0)

    for extra in range(2):
        nb = wid + extra * NW

        @pl.when(nb < NBNEW)
        def _newblk():
            for k in range(EPB // 16):
                sl = pl.ds(k * 16, 16)
                vals = plsc.load_gather(g2_v, [t3_v[nb, sl]])
                acc_v[pl.ds(N + nb * EPB + k * 16, 16)] = vals

    pltpu.sync_copy(acc_v, out_hbm.at[wid])


@functools.cache
def _make_sc_rows():
    mesh = plsc.VectorSubcoreMesh(core_axis_name="c", subcore_axis_name="s")
    return functools.partial(
        pl.kernel,
        mesh=mesh,
        compiler_params=pltpu.CompilerParams(
            needs_layout_passes=False, use_tc_tiling_on_sc=False),
        out_type=jax.ShapeDtypeStruct((2, MPAD, HID), jnp.float32),
        scratch_types=[
            pltpu.VMEM((C0_BPT, 2, EPB), jnp.int32),
            pltpu.VMEM((76, 2, EPB), jnp.int32),
            pltpu.VMEM((NBNEW, EPB), jnp.int32),
            pltpu.VMEM((EPB, HID), jnp.float32),
            pltpu.VMEM((EPB, HID), jnp.float32),
            pltpu.SemaphoreType.DMA,
            pltpu.SemaphoreType.DMA,
            pltpu.VMEM_SHARED((MPAD, HID), jnp.float32),
        ],
    )(_sc_rows_body)


def _sc_rows_body(g_hbm, e3_hbm, t3_hbm, out_hbm,
                  slab0, slab1, t3_v, rows0, rows1, sem0, sem1, acc_sh):
    c = lax.axis_index("c")
    s = lax.axis_index("s")

    @pl.when(c == 0)
    def _stage0():
        pltpu.sync_copy(e3_hbm.at[pl.ds(s * C0_BPT, C0_BPT)], slab0)
        pltpu.sync_copy(t3_hbm, t3_v)

    start1, off1, npairs1 = _core_split(s)

    @pl.when(c == 1)
    def _stage1():
        pltpu.sync_copy(e3_hbm.at[pl.ds(start1, 76)], slab1)

    def zrow(r, carry):
        for k in range(HID // 16):
            rows0[r, pl.ds(k * 16, 16)] = jnp.zeros((16,), jnp.float32)
        return carry

    lax.fori_loop(0, EPB, zrow, 0)
    base = s * RPT
    nfull = RPT // EPB
    for k in range(nfull):
        pltpu.sync_copy(rows0, acc_sh.at[pl.ds(base + k * EPB, EPB)])
    rem = RPT - nfull * EPB
    if rem:
        pltpu.sync_copy(rows0.at[pl.ds(0, rem)],
                        acc_sh.at[pl.ds(base + nfull * EPB, rem)])
    plsc.subcore_barrier()

    @pl.when(c == 0)
    def _new_edges():
        for extra in range(3):
            nb = extra * 16 + s

            @pl.when(nb < NBNEW)
            def _newblk():
                pltpu.async_copy(g_hbm.at[t3_v.at[nb]], rows0, sem0).wait()
                pltpu.sync_copy(rows0, acc_sh.at[pl.ds(N + nb * EPB, EPB)])

    bufs = (rows0, rows1)
    sems = (sem0, sem1)

    def run_pairs(slab, off, npairs):
        pltpu.async_copy(g_hbm.at[slab.at[off, 0]], rows0, sem0)
        pltpu.async_copy(g_hbm.at[slab.at[off + 1, 0]], rows1, sem1)

        def body(i, carry):
            for b in range(2):
                j = off + 2 * i + b
                buf, sem = bufs[b], sems[b]
                pltpu.make_async_copy(g_hbm.at[slab.at[j, 0]], buf, sem).wait()
                pltpu.sync_copy(buf, acc_sh.at[slab.at[j, 1]], add=True)

                @pl.when(2 * i + b + 2 < 2 * npairs)
                def _prefetch():
                    pltpu.async_copy(g_hbm.at[slab.at[j + 2, 0]], buf, sem)

            return carry

        lax.fori_loop(0, npairs, body, 0)

    @pl.when(c == 0)
    def _main0():
        run_pairs(slab0, 0, C0_BPT // 2)

    @pl.when(c == 1)
    def _main1():
        run_pairs(slab1, off1, npairs1)

    plsc.subcore_barrier()
    pltpu.sync_copy(acc_sh.at[pl.ds(base, RPT)], out_hbm.at[c, pl.ds(base, RPT)])


@functools.cache
def _make_sc_mid():
    mesh = plsc.VectorSubcoreMesh(core_axis_name="c", subcore_axis_name="s")
    return functools.partial(
        pl.kernel,
        mesh=mesh,
        compiler_params=pltpu.CompilerParams(
            needs_layout_passes=False, use_tc_tiling_on_sc=False),
        out_type=jax.ShapeDtypeStruct((MPAD,), jnp.float32),
        scratch_types=[
            pltpu.VMEM((NPT, HID), jnp.float32),
            pltpu.VMEM((NPT, HID), jnp.float32),
            pltpu.VMEM((NPT, HID), jnp.float32),
            pltpu.VMEM((NPT,), jnp.float32),
            pltpu.VMEM((NPT,), jnp.float32),
            pltpu.VMEM((HID, 16), jnp.float32),
            pltpu.VMEM((HID, 16), jnp.float32),
        ],
    )(_sc_mid_body)


def _sc_mid_body(acc_hbm, g_hbm, dinv_hbm, bc1s_hbm, wc2s_hbm, out_hbm,
                 a0_v, a1_v, g_v, dinv_v, g2_v, bc1_v, wc2_v):
    c = lax.axis_index("c")
    s = lax.axis_index("s")
    wid = s * 2 + c
    nbase = wid * NPT
    pltpu.sync_copy(acc_hbm.at[0, pl.ds(nbase, NPT)], a0_v)
    pltpu.sync_copy(acc_hbm.at[1, pl.ds(nbase, NPT)], a1_v)
    pltpu.sync_copy(g_hbm.at[pl.ds(nbase, NPT)], g_v)
    pltpu.sync_copy(dinv_hbm.at[pl.ds(nbase, NPT)], dinv_v)
    pltpu.sync_copy(bc1s_hbm, bc1_v)
    pltpu.sync_copy(wc2s_hbm, wc2_v)
    row_iota = lax.iota(jnp.int32, 16)

    def body(gg, carry):
        off = gg * 16
        dinv16 = dinv_v[pl.ds(off, 16)]
        ridx = row_iota + off
        sacc = jnp.zeros((16,), jnp.float32)
        for k in range(HID):
            cidx = jnp.full((16,), k, jnp.int32)
            col = (plsc.load_gather(a0_v, [ridx, cidx])
                   + plsc.load_gather(a1_v, [ridx, cidx])
                   + plsc.load_gather(g_v, [ridx, cidx]))
            out1 = jnp.maximum(col * dinv16 + bc1_v[k], 0.0)
            sacc = sacc + out1 * wc2_v[k]
        g2_v[pl.ds(off, 16)] = sacc * dinv16
        return carry

    lax.fori_loop(0, NPT // 16, body, 0)
    pltpu.sync_copy(g2_v, out_hbm.at[pl.ds(nbase, NPT)])


def _mlp_body(feat_ref, w1_ref, b1_ref, w2_ref, b2_ref, wf_ref, bf_ref,
              wc1_ref, gen_ref, geng_ref):
    h = jnp.dot(feat_ref[...], w1_ref[...], preferred_element_type=jnp.float32)
    h = jnp.maximum(h + b1_ref[...], 0.0)
    h = jnp.dot(h, w2_ref[...], preferred_element_type=jnp.float32)
    h = jnp.maximum(h + b2_ref[...], 0.0)
    gen = jnp.tanh(jnp.dot(h, wf_ref[...], preferred_element_type=jnp.float32)
                   + bf_ref[...])
    gen_ref[...] = gen
    wc1 = wc1_ref[...]
    parts = [
        jnp.dot(gen[:, p * D:(p + 1) * D], wc1, preferred_element_type=jnp.float32)
        for p in range(NUM_PRED)
    ]
    geng_ref[...] = jnp.concatenate(parts, axis=1)


def _deg_body(parts_ref, out_ref):
    deg = jnp.sum(parts_ref[...], axis=0, keepdims=True) + 1.0
    col = lax.broadcasted_iota(jnp.int32, (1, MPAD), 1)
    deg = jnp.where(col < N, deg, 2.0)
    out_ref[...] = lax.rsqrt(deg)


def _prep_body(x_ref, wc1_ref, gen5_ref, dinv_ref, g_ref):
    hx = jnp.dot(x_ref[...], wc1_ref[...], preferred_element_type=jnp.float32)
    g_ref[pl.ds(0, N), :] = hx * dinv_ref[pl.ds(0, N), :]
    g_ref[pl.ds(N, M - N), :] = gen5_ref[...] * dinv_ref[pl.ds(N, M - N), :]
    g_ref[pl.ds(M, MPAD - M), :] = jnp.zeros((MPAD - M, HID), jnp.float32)


def _final_body(parts_ref, g2_ref, dinv_ref, bc2_ref, out_ref):
    acc2 = jnp.sum(parts_ref[...], axis=0, keepdims=True)
    out2 = dinv_ref[...] * (acc2 + g2_ref[...]) + bc2_ref[...]
    out_ref[...] = 1.0 / (1.0 + jnp.exp(-out2))


def kernel(feat, x, edge_index, tails, W1, b1, W2, b2, Wf, bf, Wc1, bc1, Wc2, bc2):
    e3 = edge_index.reshape(2, NB_E, EPB).transpose(1, 0, 2)
    tails_rep = jnp.broadcast_to(tails[:, None], (T, NUM_PRED)).reshape(-1)
    t3 = jnp.concatenate(
        [tails_rep, jnp.zeros((TPAD - T * NUM_PRED,), jnp.int32)]
    ).reshape(NBNEW, EPB)

    parts = _make_sc_count()(e3)

    dinv_row = pl.pallas_call(
        _deg_body,
        out_shape=jax.ShapeDtypeStruct((1, MPAD), jnp.float32),
    )(parts)
    dinv_col = dinv_row.reshape(MPAD, 1)

    gen_feat, geng = pl.pallas_call(
        _mlp_body,
        out_shape=(
            jax.ShapeDtypeStruct((T, NUM_PRED * D), jnp.float32),
            jax.ShapeDtypeStruct((T, NUM_PRED * HID), jnp.float32),
        ),
    )(feat, W1, b1.reshape(1, -1), W2, b2.reshape(1, -1), Wf, bf.reshape(1, -1),
      Wc1)

    gen5 = geng.reshape(T * NUM_PRED, HID)

    g = pl.pallas_call(
        _prep_body,
        out_shape=jax.ShapeDtypeStruct((MPAD, HID), jnp.float32),
    )(x, Wc1, gen5, dinv_col)

    acc = _make_sc_rows()(g, e3, t3)

    dinv_flat = dinv_row.reshape(MPAD)
    bc1s = jnp.broadcast_to(bc1[:, None], (HID, 16))
    wc2s = jnp.broadcast_to(Wc2[:, 0][:, None], (HID, 16))
    g2 = _make_sc_mid()(acc, g, dinv_flat, bc1s, wc2s)

    parts2 = _make_sc_scalar()(g2, e3, t3)

    pred_row = pl.pallas_call(
        _final_body,
        out_shape=jax.ShapeDtypeStruct((1, MPAD), jnp.float32),
    )(parts2, g2.reshape(1, MPAD), dinv_row, bc2.reshape(1, 1))

    class_pred = pred_row.reshape(MPAD, 1)[:M]
    return (gen_feat, class_pred)


# speedup vs baseline: 1.1585x; 1.1585x over previous
import functools

import jax
import jax.numpy as jnp
from jax import lax
from jax.experimental import pallas as pl
from jax.experimental.pallas import tpu as pltpu
from jax.experimental.pallas import tpu_sc as plsc

N = 10000
E = 320000
D = 128
T = 1000
NUM_PRED = 5
HID = 64
M = N + T * NUM_PRED
MPAD = 15360
NW = 32
EPB = 128
NB_E = E // EPB
TPAD = 5120
NBNEW = TPAD // EPB
RPT = MPAD // 16
NPT = MPAD // NW

C0_BPT = 86
C1_BASE = 1376

EV_BASE = 78
EV_SLAB = 79


def _core_split(s):
    lt2 = s < 2
    start = jnp.where(lt2, C1_BASE + s * 72, C1_BASE + 144 + (s - 2) * 70)
    off = jnp.where(lt2, 0, 2)
    npairs = jnp.where(lt2, 36, 35)
    return start - off, off, npairs


def _even_split(wid):
    start = wid * EV_BASE + jnp.minimum(wid, 4)
    cnt = EV_BASE + jnp.where(wid < 4, 1, 0)
    start_copy = jnp.minimum(start, NB_E - EV_SLAB)
    return start_copy, start - start_copy, cnt


@functools.cache
def _make_sc_count():
    mesh = plsc.VectorSubcoreMesh(core_axis_name="c", subcore_axis_name="s")
    return functools.partial(
        pl.kernel,
        mesh=mesh,
        compiler_params=pltpu.CompilerParams(
            needs_layout_passes=False, use_tc_tiling_on_sc=False),
        out_type=jax.ShapeDtypeStruct((NW, MPAD), jnp.float32),
        scratch_types=[
            pltpu.VMEM((EV_SLAB, 2, EPB), jnp.int32),
            pltpu.VMEM((MPAD,), jnp.float32),
        ],
    )(_sc_count_body)


def _sc_count_body(e3_hbm, out_hbm, slab, acc_v):
    c = lax.axis_index("c")
    s = lax.axis_index("s")
    wid = s * 2 + c
    start_copy, off, cnt = _even_split(wid)
    pltpu.sync_copy(e3_hbm.at[pl.ds(start_copy, EV_SLAB)], slab)

    def zero(i, carry):
        acc_v[pl.ds(i * 16, 16)] = jnp.zeros((16,), jnp.float32)
        return carry

    lax.fori_loop(0, MPAD // 16, zero, 0)
    ones = jnp.full((16,), 1.0, jnp.float32)

    def body(i, carry):
        j = off + i
        for k in range(EPB // 16):
            plsc.addupdate_scatter(acc_v, [slab[j, 1, pl.ds(k * 16, 16)]], ones)
        return carry

    lax.fori_loop(0, cnt, body, 0)
    pltpu.sync_copy(acc_v, out_hbm.at[wid])


@functools.cache
def _make_sc_scalar():
    mesh = plsc.VectorSubcoreMesh(core_axis_name="c", subcore_axis_name="s")
    return functools.partial(
        pl.kernel,
        mesh=mesh,
        compiler_params=pltpu.CompilerParams(
            needs_layout_passes=False, use_tc_tiling_on_sc=False),
        out_type=jax.ShapeDtypeStruct((NW, MPAD), jnp.float32),
        scratch_types=[
            pltpu.VMEM((EV_SLAB, 2, EPB), jnp.int32),
            pltpu.VMEM((NBNEW, EPB), jnp.int32),
            pltpu.VMEM((MPAD,), jnp.float32),
            pltpu.VMEM((MPAD,), jnp.float32),
        ],
    )(_sc_scalar_body)


def _sc_scalar_body(g2_hbm, e3_hbm, t3_hbm, out_hbm, slab, t3_v, g2_v, acc_v):
    c = lax.axis_index("c")
    s = lax.axis_index("s")
    wid = s * 2 + c
    start_copy, off, cnt = _even_split(wid)
    pltpu.sync_copy(e3_hbm.at[pl.ds(start_copy, EV_SLAB)], slab)
    pltpu.sync_copy(t3_hbm, t3_v)
    pltpu.sync_copy(g2_hbm, g2_v)

    def zero(i, carry):
        acc_v[pl.ds(i * 16, 16)] = jnp.zeros((16,), jnp.float32)
        return carry

    lax.fori_loop(0, MPAD // 16, zero, 0)

    def body(i, carry):
        j = off + i
        for k in range(EPB // 16):
            sl = pl.ds(k * 16, 16)
            vals = plsc.load_gather(g2_v, [slab[j, 0, sl]])
            plsc.addupdate_scatter(acc_v, [slab[j, 1, sl]], vals)
        return carry

    lax.fori_loop(0, cnt, body, 0)

    for extra in range(2):
        nb = wid + extra * NW

        @pl.when(nb < NBNEW)
        def _newblk():
            for k in range(EPB // 16):
                sl = pl.ds(k * 16, 16)
                vals = plsc.load_gather(g2_v, [t3_v[nb, sl]])
                acc_v[pl.ds(N + nb * EPB + k * 16, 16)] = vals

    pltpu.sync_copy(acc_v, out_hbm.at[wid])


@functools.cache
def _make_sc_rows():
    mesh = plsc.VectorSubcoreMesh(core_axis_name="c", subcore_axis_name="s")
    return functools.partial(
        pl.kernel,
        mesh=mesh,
        compiler_params=pltpu.CompilerParams(
            needs_layout_passes=False, use_tc_tiling_on_sc=False),
        out_type=jax.ShapeDtypeStruct((2, MPAD, HID), jnp.float32),
        scratch_types=[
            pltpu.VMEM((C0_BPT, 2, EPB), jnp.int32),
            pltpu.VMEM((72, 2, EPB), jnp.int32),
            pltpu.VMEM((NBNEW, EPB), jnp.int32),
            pltpu.VMEM((EPB, HID), jnp.float32),
            pltpu.VMEM((EPB, HID), jnp.float32),
            pltpu.SemaphoreType.DMA,
            pltpu.SemaphoreType.DMA,
            pltpu.VMEM_SHARED((MPAD, HID), jnp.float32),
        ],
    )(_sc_rows_body)


def _sc_rows_body(g_hbm, e3_hbm, t3_hbm, out_hbm,
                  slab0, slab1, t3_v, rows0, rows1, sem0, sem1, acc_sh):
    c = lax.axis_index("c")
    s = lax.axis_index("s")

    @pl.when(c == 0)
    def _stage0():
        pltpu.sync_copy(e3_hbm.at[pl.ds(s * C0_BPT, C0_BPT)], slab0)
        pltpu.sync_copy(t3_hbm, t3_v)

    start1, off1, npairs1 = _core_split(s)

    @pl.when(c == 1)
    def _stage1():
        pltpu.sync_copy(e3_hbm.at[pl.ds(start1, 72)], slab1)

    def zrow(r, carry):
        for k in range(HID // 16):
            rows0[r, pl.ds(k * 16, 16)] = jnp.zeros((16,), jnp.float32)
        return carry

    lax.fori_loop(0, EPB, zrow, 0)
    base = s * RPT
    nfull = RPT // EPB
    for k in range(nfull):
        pltpu.sync_copy(rows0, acc_sh.at[pl.ds(base + k * EPB, EPB)])
    rem = RPT - nfull * EPB
    if rem:
        pltpu.sync_copy(rows0.at[pl.ds(0, rem)],
                        acc_sh.at[pl.ds(base + nfull * EPB, rem)])
    plsc.subcore_barrier()

    @pl.when(c == 0)
    def _new_edges():
        for extra in range(3):
            nb = extra * 16 + s

            @pl.when(nb < NBNEW)
            def _newblk():
                pltpu.async_copy(g_hbm.at[t3_v.at[nb]], rows0, sem0).wait()
                pltpu.sync_copy(rows0, acc_sh.at[pl.ds(N + nb * EPB, EPB)])

    bufs = (rows0, rows1)
    sems = (sem0, sem1)

    def run_pairs(slab, off, npairs):
        pltpu.async_copy(g_hbm.at[slab.at[off, 0]], rows0, sem0)
        pltpu.async_copy(g_hbm.at[slab.at[off + 1, 0]], rows1, sem1)

        def body(i, carry):
            for b in range(2):
                j = off + 2 * i + b
                buf, sem = bufs[b], sems[b]
                pltpu.make_async_copy(g_hbm.at[slab.at[j, 0]], buf, sem).wait()
                pltpu.sync_copy(buf, acc_sh.at[slab.at[j, 1]], add=True)

                @pl.when(2 * i + b + 2 < 2 * npairs)
                def _prefetch():
                    pltpu.async_copy(g_hbm.at[slab.at[j + 2, 0]], buf, sem)

            return carry

        lax.fori_loop(0, npairs, body, 0)

    @pl.when(c == 0)
    def _main0():
        run_pairs(slab0, 0, C0_BPT // 2)

    @pl.when(c == 1)
    def _main1():
        run_pairs(slab1, off1, npairs1)

    plsc.subcore_barrier()
    pltpu.sync_copy(acc_sh.at[pl.ds(base, RPT)], out_hbm.at[c, pl.ds(base, RPT)])


@functools.cache
def _make_sc_mid():
    mesh = plsc.VectorSubcoreMesh(core_axis_name="c", subcore_axis_name="s")
    return functools.partial(
        pl.kernel,
        mesh=mesh,
        compiler_params=pltpu.CompilerParams(
            needs_layout_passes=False, use_tc_tiling_on_sc=False),
        out_type=jax.ShapeDtypeStruct((MPAD,), jnp.float32),
        scratch_types=[
            pltpu.VMEM((NPT, HID), jnp.float32),
            pltpu.VMEM((NPT, HID), jnp.float32),
            pltpu.VMEM((NPT, HID), jnp.float32),
            pltpu.VMEM((NPT,), jnp.float32),
            pltpu.VMEM((NPT,), jnp.float32),
            pltpu.VMEM((HID, 16), jnp.float32),
            pltpu.VMEM((HID, 16), jnp.float32),
        ],
    )(_sc_mid_body)


def _sc_mid_body(acc_hbm, g_hbm, dinv_hbm, bc1s_hbm, wc2s_hbm, out_hbm,
                 a0_v, a1_v, g_v, dinv_v, g2_v, bc1_v, wc2_v):
    c = lax.axis_index("c")
    s = lax.axis_index("s")
    wid = s * 2 + c
    nbase = wid * NPT
    pltpu.sync_copy(acc_hbm.at[0, pl.ds(nbase, NPT)], a0_v)
    pltpu.sync_copy(acc_hbm.at[1, pl.ds(nbase, NPT)], a1_v)
    pltpu.sync_copy(g_hbm.at[pl.ds(nbase, NPT)], g_v)
    pltpu.sync_copy(dinv_hbm.at[pl.ds(nbase, NPT)], dinv_v)
    pltpu.sync_copy(bc1s_hbm, bc1_v)
    pltpu.sync_copy(wc2s_hbm, wc2_v)
    row_iota = lax.iota(jnp.int32, 16)

    def tbody(r, carry):
        for k in range(HID // 16):
            sl = pl.ds(k * 16, 16)
            a0_v[r, sl] = a0_v[r, sl] + a1_v[r, sl] + g_v[r, sl]
        return carry

    lax.fori_loop(0, NPT, tbody, 0)

    def body(gg, carry):
        off = gg * 16
        dinv16 = dinv_v[pl.ds(off, 16)]
        ridx = row_iota + off
        saccs = [jnp.zeros((16,), jnp.float32) for _ in range(4)]
        for k0 in range(0, HID, 4):
            for u in range(4):
                k = k0 + u
                cidx = jnp.full((16,), k, jnp.int32)
                col = plsc.load_gather(a0_v, [ridx, cidx])
                out1 = jnp.maximum(col * dinv16 + bc1_v[k], 0.0)
                saccs[u] = saccs[u] + out1 * wc2_v[k]
        sacc = (saccs[0] + saccs[1]) + (saccs[2] + saccs[3])
        g2_v[pl.ds(off, 16)] = sacc * dinv16
        return carry

    lax.fori_loop(0, NPT // 16, body, 0)
    pltpu.sync_copy(g2_v, out_hbm.at[pl.ds(nbase, NPT)])


def _mlp_body(feat_ref, w1_ref, b1_ref, w2_ref, b2_ref, wf_ref, bf_ref,
              wc1_ref, gen_ref, geng_ref):
    h = jnp.dot(feat_ref[...], w1_ref[...], preferred_element_type=jnp.float32)
    h = jnp.maximum(h + b1_ref[...], 0.0)
    h = jnp.dot(h, w2_ref[...], preferred_element_type=jnp.float32)
    h = jnp.maximum(h + b2_ref[...], 0.0)
    gen = jnp.tanh(jnp.dot(h, wf_ref[...], preferred_element_type=jnp.float32)
                   + bf_ref[...])
    gen_ref[...] = gen
    wc1 = wc1_ref[...]
    parts = [
        jnp.dot(gen[:, p * D:(p + 1) * D], wc1, preferred_element_type=jnp.float32)
        for p in range(NUM_PRED)
    ]
    geng_ref[...] = jnp.concatenate(parts, axis=1)


def _deg_body(parts_ref, out_ref):
    deg = jnp.sum(parts_ref[...], axis=0, keepdims=True) + 1.0
    col = lax.broadcasted_iota(jnp.int32, (1, MPAD), 1)
    deg = jnp.where(col < N, deg, 2.0)
    out_ref[...] = lax.rsqrt(deg)


def _prep_body(x_ref, wc1_ref, gen5_ref, dinv_ref, g_ref):
    hx = jnp.dot(x_ref[...], wc1_ref[...], preferred_element_type=jnp.float32)
    g_ref[pl.ds(0, N), :] = hx * dinv_ref[pl.ds(0, N), :]
    g_ref[pl.ds(N, M - N), :] = gen5_ref[...] * dinv_ref[pl.ds(N, M - N), :]
    g_ref[pl.ds(M, MPAD - M), :] = jnp.zeros((MPAD - M, HID), jnp.float32)


def _final_body(parts_ref, g2_ref, dinv_ref, bc2_ref, out_ref):
    acc2 = jnp.sum(parts_ref[...], axis=0, keepdims=True)
    out2 = dinv_ref[...] * (acc2 + g2_ref[...]) + bc2_ref[...]
    out_ref[...] = 1.0 / (1.0 + jnp.exp(-out2))


def kernel(feat, x, edge_index, tails, W1, b1, W2, b2, Wf, bf, Wc1, bc1, Wc2, bc2):
    e3 = edge_index.reshape(2, NB_E, EPB).transpose(1, 0, 2)
    tails_rep = jnp.broadcast_to(tails[:, None], (T, NUM_PRED)).reshape(-1)
    t3 = jnp.concatenate(
        [tails_rep, jnp.zeros((TPAD - T * NUM_PRED,), jnp.int32)]
    ).reshape(NBNEW, EPB)

    parts = _make_sc_count()(e3)

    dinv_row = pl.pallas_call(
        _deg_body,
        out_shape=jax.ShapeDtypeStruct((1, MPAD), jnp.float32),
    )(parts)
    dinv_col = dinv_row.reshape(MPAD, 1)

    gen_feat, geng = pl.pallas_call(
        _mlp_body,
        out_shape=(
            jax.ShapeDtypeStruct((T, NUM_PRED * D), jnp.float32),
            jax.ShapeDtypeStruct((T, NUM_PRED * HID), jnp.float32),
        ),
    )(feat, W1, b1.reshape(1, -1), W2, b2.reshape(1, -1), Wf, bf.reshape(1, -1),
      Wc1)

    gen5 = geng.reshape(T * NUM_PRED, HID)

    g = pl.pallas_call(
        _prep_body,
        out_shape=jax.ShapeDtypeStruct((MPAD, HID), jnp.float32),
    )(x, Wc1, gen5, dinv_col)

    acc = _make_sc_rows()(g, e3, t3)

    dinv_flat = dinv_row.reshape(MPAD)
    bc1s = jnp.broadcast_to(bc1[:, None], (HID, 16))
    wc2s = jnp.broadcast_to(Wc2[:, 0][:, None], (HID, 16))
    g2 = _make_sc_mid()(acc, g, dinv_flat, bc1s, wc2s)

    parts2 = _make_sc_scalar()(g2, e3, t3)

    pred_row = pl.pallas_call(
        _final_body,
        out_shape=jax.ShapeDtypeStruct((1, MPAD), jnp.float32),
    )(parts2, g2.reshape(1, MPAD), dinv_row, bc2.reshape(1, 1))

    class_pred = pred_row.reshape(MPAD, 1)[:M]
    return (gen_feat, class_pred)


# speedup vs baseline: 1.2329x; 1.0642x over previous
import functools

import jax
import jax.numpy as jnp
from jax import lax
from jax.experimental import pallas as pl
from jax.experimental.pallas import tpu as pltpu
from jax.experimental.pallas import tpu_sc as plsc

N = 10000
E = 320000
D = 128
T = 1000
NUM_PRED = 5
HID = 64
M = N + T * NUM_PRED
MPAD = 15360
NW = 32
EPB = 128
NB_E = E // EPB
TPAD = 5120
NBNEW = TPAD // EPB
RPT = MPAD // 16
NPT = MPAD // NW

C0_SLAB = 78
C1_SLAB = 80
C1_BASE = 1230

EV_BASE = 78
EV_SLAB = 79


def _core0_split(s):
    lt = s < 7
    start = jnp.where(lt, s * 78, 546 + (s - 7) * 76)
    npairs = jnp.where(lt, 39, 38)
    start_copy = jnp.minimum(start, 1230 - C0_SLAB)
    return start_copy, start - start_copy, npairs


def _core1_split(s):
    lt = s < 11
    start = jnp.where(lt, C1_BASE + s * 80, C1_BASE + 880 + (s - 11) * 78)
    npairs = jnp.where(lt, 40, 39)
    start_copy = jnp.minimum(start, NB_E - C1_SLAB)
    return start_copy, start - start_copy, npairs


def _even_split(wid):
    start = wid * EV_BASE + jnp.minimum(wid, 4)
    cnt = EV_BASE + jnp.where(wid < 4, 1, 0)
    start_copy = jnp.minimum(start, NB_E - EV_SLAB)
    return start_copy, start - start_copy, cnt


@functools.cache
def _make_sc_count():
    mesh = plsc.VectorSubcoreMesh(core_axis_name="c", subcore_axis_name="s")
    return functools.partial(
        pl.kernel,
        mesh=mesh,
        compiler_params=pltpu.CompilerParams(
            needs_layout_passes=False, use_tc_tiling_on_sc=False),
        out_type=jax.ShapeDtypeStruct((NW, MPAD), jnp.float32),
        scratch_types=[
            pltpu.VMEM((EV_SLAB, 2, EPB), jnp.int32),
            pltpu.VMEM((MPAD,), jnp.float32),
        ],
    )(_sc_count_body)


def _sc_count_body(e3_hbm, out_hbm, slab, acc_v):
    c = lax.axis_index("c")
    s = lax.axis_index("s")
    wid = s * 2 + c
    start_copy, off, cnt = _even_split(wid)
    pltpu.sync_copy(e3_hbm.at[pl.ds(start_copy, EV_SLAB)], slab)

    def zero(i, carry):
        acc_v[pl.ds(i * 16, 16)] = jnp.zeros((16,), jnp.float32)
        return carry

    lax.fori_loop(0, MPAD // 16, zero, 0)
    ones = jnp.full((16,), 1.0, jnp.float32)

    def body(i, carry):
        j = off + i
        for k in range(EPB // 16):
            plsc.addupdate_scatter(acc_v, [slab[j, 1, pl.ds(k * 16, 16)]], ones)
        return carry

    lax.fori_loop(0, cnt, body, 0)
    pltpu.sync_copy(acc_v, out_hbm.at[wid])


@functools.cache
def _make_sc_scalar():
    mesh = plsc.VectorSubcoreMesh(core_axis_name="c", subcore_axis_name="s")
    return functools.partial(
        pl.kernel,
        mesh=mesh,
        compiler_params=pltpu.CompilerParams(
            needs_layout_passes=False, use_tc_tiling_on_sc=False),
        out_type=jax.ShapeDtypeStruct((NW, MPAD), jnp.float32),
        scratch_types=[
            pltpu.VMEM((EV_SLAB, 2, EPB), jnp.int32),
            pltpu.VMEM((NBNEW, EPB), jnp.int32),
            pltpu.VMEM((MPAD,), jnp.float32),
            pltpu.VMEM((MPAD,), jnp.float32),
        ],
    )(_sc_scalar_body)


def _sc_scalar_body(g2_hbm, e3_hbm, t3_hbm, out_hbm, slab, t3_v, g2_v, acc_v):
    c = lax.axis_index("c")
    s = lax.axis_index("s")
    wid = s * 2 + c
    start_copy, off, cnt = _even_split(wid)
    pltpu.sync_copy(e3_hbm.at[pl.ds(start_copy, EV_SLAB)], slab)
    pltpu.sync_copy(t3_hbm, t3_v)
    pltpu.sync_copy(g2_hbm, g2_v)

    def zero(i, carry):
        acc_v[pl.ds(i * 16, 16)] = jnp.zeros((16,), jnp.float32)
        return carry

    lax.fori_loop(0, MPAD // 16, zero, 0)

    def body(i, carry):
        j = off + i
        for k in range(EPB // 16):
            sl = pl.ds(k * 16, 16)
            vals = plsc.load_gather(g2_v, [slab[j, 0, sl]])
            plsc.addupdate_scatter(acc_v, [slab[j, 1, sl]], vals)
        return carry

    lax.fori_loop(0, cnt, body, 0)

    for extra in range(2):
        nb = wid + extra * NW

        @pl.when(nb < NBNEW)
        def _newblk():
            for k in range(EPB // 16):
                sl = pl.ds(k * 16, 16)
                vals = plsc.load_gather(g2_v, [t3_v[nb, sl]])
                acc_v[pl.ds(N + nb * EPB + k * 16, 16)] = vals

    pltpu.sync_copy(acc_v, out_hbm.at[wid])


@functools.cache
def _make_sc_rows():
    mesh = plsc.VectorSubcoreMesh(core_axis_name="c", subcore_axis_name="s")
    return functools.partial(
        pl.kernel,
        mesh=mesh,
        compiler_params=pltpu.CompilerParams(
            needs_layout_passes=False, use_tc_tiling_on_sc=False),
        out_type=jax.ShapeDtypeStruct((2, MPAD, HID), jnp.float32),
        scratch_types=[
            pltpu.VMEM((C0_SLAB, 2, EPB), jnp.int32),
            pltpu.VMEM((C1_SLAB, 2, EPB), jnp.int32),
            pltpu.VMEM((NBNEW, EPB), jnp.int32),
            pltpu.VMEM((EPB, HID), jnp.float32),
            pltpu.VMEM((EPB, HID), jnp.float32),
            pltpu.SemaphoreType.DMA,
            pltpu.SemaphoreType.DMA,
            pltpu.VMEM_SHARED((MPAD, HID), jnp.float32),
        ],
    )(_sc_rows_body)


def _sc_rows_body(g_hbm, e3_hbm, t3_hbm, out_hbm,
                  slab0, slab1, t3_v, rows0, rows1, sem0, sem1, acc_sh):
    c = lax.axis_index("c")
    s = lax.axis_index("s")

    start0, off0, npairs0 = _core0_split(s)
    start1, off1, npairs1 = _core1_split(s)

    @pl.when(c == 0)
    def _stage0():
        pltpu.sync_copy(e3_hbm.at[pl.ds(start0, C0_SLAB)], slab0)
        pltpu.sync_copy(t3_hbm, t3_v)

    @pl.when(c == 1)
    def _stage1():
        pltpu.sync_copy(e3_hbm.at[pl.ds(start1, C1_SLAB)], slab1)

    def zrow(r, carry):
        for k in range(HID // 16):
            rows0[r, pl.ds(k * 16, 16)] = jnp.zeros((16,), jnp.float32)
        return carry

    lax.fori_loop(0, EPB, zrow, 0)
    base = s * RPT
    nfull = RPT // EPB
    for k in range(nfull):
        pltpu.sync_copy(rows0, acc_sh.at[pl.ds(base + k * EPB, EPB)])
    rem = RPT - nfull * EPB
    if rem:
        pltpu.sync_copy(rows0.at[pl.ds(0, rem)],
                        acc_sh.at[pl.ds(base + nfull * EPB, rem)])
    plsc.subcore_barrier()

    @pl.when(c == 0)
    def _new_edges():
        for extra in range(3):
            nb = extra * 16 + s

            @pl.when(nb < NBNEW)
            def _newblk():
                pltpu.async_copy(g_hbm.at[t3_v.at[nb]], rows0, sem0).wait()
                pltpu.sync_copy(rows0, acc_sh.at[pl.ds(N + nb * EPB, EPB)])

    bufs = (rows0, rows1)
    sems = (sem0, sem1)

    def run_pairs(slab, off, npairs):
        pltpu.async_copy(g_hbm.at[slab.at[off, 0]], rows0, sem0)
        pltpu.async_copy(g_hbm.at[slab.at[off + 1, 0]], rows1, sem1)

        def body(i, carry):
            for b in range(2):
                j = off + 2 * i + b
                buf, sem = bufs[b], sems[b]
                pltpu.make_async_copy(g_hbm.at[slab.at[j, 0]], buf, sem).wait()
                pltpu.sync_copy(buf, acc_sh.at[slab.at[j, 1]], add=True)

                @pl.when(2 * i + b + 2 < 2 * npairs)
                def _prefetch():
                    pltpu.async_copy(g_hbm.at[slab.at[j + 2, 0]], buf, sem)

            return carry

        lax.fori_loop(0, npairs, body, 0)

    @pl.when(c == 0)
    def _main0():
        run_pairs(slab0, off0, npairs0)

    @pl.when(c == 1)
    def _main1():
        run_pairs(slab1, off1, npairs1)

    plsc.subcore_barrier()
    pltpu.sync_copy(acc_sh.at[pl.ds(base, RPT)], out_hbm.at[c, pl.ds(base, RPT)])


@functools.cache
def _make_sc_mid():
    mesh = plsc.VectorSubcoreMesh(core_axis_name="c", subcore_axis_name="s")
    return functools.partial(
        pl.kernel,
        mesh=mesh,
        compiler_params=pltpu.CompilerParams(
            needs_layout_passes=False, use_tc_tiling_on_sc=False),
        out_type=jax.ShapeDtypeStruct((MPAD,), jnp.float32),
        scratch_types=[
            pltpu.VMEM((NPT, HID), jnp.float32),
            pltpu.VMEM((NPT, HID), jnp.float32),
            pltpu.VMEM((NPT, HID), jnp.float32),
            pltpu.VMEM((NPT,), jnp.float32),
            pltpu.VMEM((NPT,), jnp.float32),
            pltpu.VMEM((HID, 16), jnp.float32),
            pltpu.VMEM((HID, 16), jnp.float32),
        ],
    )(_sc_mid_body)


def _sc_mid_body(acc_hbm, g_hbm, dinv_hbm, bc1s_hbm, wc2s_hbm, out_hbm,
                 a0_v, a1_v, g_v, dinv_v, g2_v, bc1_v, wc2_v):
    c = lax.axis_index("c")
    s = lax.axis_index("s")
    wid = s * 2 + c
    nbase = wid * NPT
    pltpu.sync_copy(acc_hbm.at[0, pl.ds(nbase, NPT)], a0_v)
    pltpu.sync_copy(acc_hbm.at[1, pl.ds(nbase, NPT)], a1_v)
    pltpu.sync_copy(g_hbm.at[pl.ds(nbase, NPT)], g_v)
    pltpu.sync_copy(dinv_hbm.at[pl.ds(nbase, NPT)], dinv_v)
    pltpu.sync_copy(bc1s_hbm, bc1_v)
    pltpu.sync_copy(wc2s_hbm, wc2_v)
    row_iota = lax.iota(jnp.int32, 16)

    def tbody(r, carry):
        for k in range(HID // 16):
            sl = pl.ds(k * 16, 16)
            a0_v[r, sl] = a0_v[r, sl] + a1_v[r, sl] + g_v[r, sl]
        return carry

    lax.fori_loop(0, NPT, tbody, 0)

    def body(gg, carry):
        off = gg * 16
        dinv16 = dinv_v[pl.ds(off, 16)]
        ridx = row_iota + off
        saccs = [jnp.zeros((16,), jnp.float32) for _ in range(4)]
        for k0 in range(0, HID, 4):
            for u in range(4):
                k = k0 + u
                cidx = jnp.full((16,), k, jnp.int32)
                col = plsc.load_gather(a0_v, [ridx, cidx])
                out1 = jnp.maximum(col * dinv16 + bc1_v[k], 0.0)
                saccs[u] = saccs[u] + out1 * wc2_v[k]
        sacc = (saccs[0] + saccs[1]) + (saccs[2] + saccs[3])
        g2_v[pl.ds(off, 16)] = sacc * dinv16
        return carry

    lax.fori_loop(0, NPT // 16, body, 0)
    pltpu.sync_copy(g2_v, out_hbm.at[pl.ds(nbase, NPT)])


def _mlp_body(feat_ref, w1_ref, b1_ref, w2_ref, b2_ref, wf_ref, bf_ref,
              wc1_ref, gen_ref, geng_ref):
    h = jnp.dot(feat_ref[...], w1_ref[...], preferred_element_type=jnp.float32)
    h = jnp.maximum(h + b1_ref[...], 0.0)
    h = jnp.dot(h, w2_ref[...], preferred_element_type=jnp.float32)
    h = jnp.maximum(h + b2_ref[...], 0.0)
    gen = jnp.tanh(jnp.dot(h, wf_ref[...], preferred_element_type=jnp.float32)
                   + bf_ref[...])
    gen_ref[...] = gen
    wc1 = wc1_ref[...]
    parts = [
        jnp.dot(gen[:, p * D:(p + 1) * D], wc1, preferred_element_type=jnp.float32)
        for p in range(NUM_PRED)
    ]
    geng_ref[...] = jnp.concatenate(parts, axis=1)


def _deg_body(parts_ref, out_ref):
    deg = jnp.sum(parts_ref[...], axis=0, keepdims=True) + 1.0
    col = lax.broadcasted_iota(jnp.int32, (1, MPAD), 1)
    deg = jnp.where(col < N, deg, 2.0)
    out_ref[...] = lax.rsqrt(deg)


def _prep_body(x_ref, wc1_ref, gen5_ref, dinv_ref, g_ref):
    dinv_col = jnp.transpose(dinv_ref[...])
    hx = jnp.dot(x_ref[...], wc1_ref[...], preferred_element_type=jnp.float32)
    g_ref[pl.ds(0, N), :] = hx * dinv_col[0:N, :]
    g_ref[pl.ds(N, M - N), :] = gen5_ref[...] * dinv_col[N:M, :]
    g_ref[pl.ds(M, MPAD - M), :] = jnp.zeros((MPAD - M, HID), jnp.float32)


def _final_body(parts_ref, g2_ref, dinv_ref, bc2_ref, out_ref):
    acc2 = jnp.sum(parts_ref[...], axis=0, keepdims=True)
    out2 = dinv_ref[...] * (acc2 + g2_ref[...]) + bc2_ref[...]
    out_ref[...] = 1.0 / (1.0 + jnp.exp(-out2))


def kernel(feat, x, edge_index, tails, W1, b1, W2, b2, Wf, bf, Wc1, bc1, Wc2, bc2):
    e3 = edge_index.reshape(2, NB_E, EPB).transpose(1, 0, 2)
    tails_rep = jnp.broadcast_to(tails[:, None], (T, NUM_PRED)).reshape(-1)
    t3 = jnp.concatenate(
        [tails_rep, jnp.zeros((TPAD - T * NUM_PRED,), jnp.int32)]
    ).reshape(NBNEW, EPB)

    parts = _make_sc_count()(e3)

    dinv_row = pl.pallas_call(
        _deg_body,
        out_shape=jax.ShapeDtypeStruct((1, MPAD), jnp.float32),
    )(parts)

    gen_feat, geng = pl.pallas_call(
        _mlp_body,
        out_shape=(
            jax.ShapeDtypeStruct((T, NUM_PRED * D), jnp.float32),
            jax.ShapeDtypeStruct((T, NUM_PRED * HID), jnp.float32),
        ),
    )(feat, W1, b1.reshape(1, -1), W2, b2.reshape(1, -1), Wf, bf.reshape(1, -1),
      Wc1)

    gen5 = geng.reshape(T * NUM_PRED, HID)

    g = pl.pallas_call(
        _prep_body,
        out_shape=jax.ShapeDtypeStruct((MPAD, HID), jnp.float32),
    )(x, Wc1, gen5, dinv_row)

    acc = _make_sc_rows()(g, e3, t3)

    dinv_flat = dinv_row.reshape(MPAD)
    bc1s = jnp.broadcast_to(bc1[:, None], (HID, 16))
    wc2s = jnp.broadcast_to(Wc2[:, 0][:, None], (HID, 16))
    g2 = _make_sc_mid()(acc, g, dinv_flat, bc1s, wc2s)

    parts2 = _make_sc_scalar()(g2, e3, t3)

    pred_row = pl.pallas_call(
        _final_body,
        out_shape=jax.ShapeDtypeStruct((1, MPAD), jnp.float32),
    )(parts2, g2.reshape(1, MPAD), dinv_row, bc2.reshape(1, 1))

    class_pred = pred_row.reshape(MPAD, 1)[:M]
    return (gen_feat, class_pred)
